# baseline (speedup 1.0000x reference)
import functools

import jax
import jax.numpy as jnp
from jax import lax
from jax.experimental import pallas as pl
from jax.experimental.pallas import tpu as pltpu
from jax.experimental.pallas import tpu_sc as plsc

B, H, W, C = 8, 384, 384, 96
HW = H * W
NC, NS = 2, 16
WPB = (NC * NS) // B
RW = HW // WPB
R = 256
CH = R * C
NCHUNK = RW // R
NB = 4
G = C // 16


def _argmax_body(x_hbm, out_hbm, b0, b1, b2, b3, pval, pidx, shval, shidx,
                 cmbv, cmbi, obuf, s0, s1, s2, s3):
    bufs = (b0, b1, b2, b3)
    sems = (s0, s1, s2, s3)
    c = lax.axis_index("c")
    s = lax.axis_index("s")
    b = c * (B // NC) + s // WPB
    q = s % WPB
    spat_base = q * RW
    base_off = (b * HW + spat_base) * C

    def start(k, j):
        off = base_off + k * CH
        pltpu.make_async_copy(x_hbm.at[pl.ds(off, CH)], bufs[j], sems[j]).start()

    def wait(j):
        pltpu.make_async_copy(x_hbm.at[pl.ds(0, CH)], bufs[j], sems[j]).wait()

    for j in range(NB):
        start(j, j)

    ninf = jnp.full((16,), -jnp.inf, jnp.float32)
    zero = jnp.zeros((16,), jnp.int32)
    state = tuple([ninf] * G + [zero] * G)

    def chunk_rows(buf):
        def row_body(r, carry):
            rvec = carry[0]
            bvs = list(carry[1:1 + G])
            bis = list(carry[1 + G:])
            base = r * C
            for g in range(G):
                v = buf[pl.ds(base + g * 16, 16)]
                m = v > bvs[g]
                bvs[g] = jnp.where(m, v, bvs[g])
                bis[g] = jnp.where(m, rvec, bis[g])
            return (rvec + 1,) + tuple(bvs) + tuple(bis)
        return row_body

    def outer(t, state):
        for j in range(NB):
            k = t * NB + j
            wait(j)
            rvec0 = zero + (spat_base + k * R)
            carry = (rvec0,) + state
            carry = lax.fori_loop(0, R, chunk_rows(bufs[j]), carry,
                                  unroll=2)
            state = carry[1:]

            @pl.when(k + NB < NCHUNK)
            def _():
                start(k + NB, j)
        return state

    state = lax.fori_loop(0, NCHUNK // NB, outer, state)
    bvs = state[:G]
    bis = state[G:]

    for g in range(G):
        pval[pl.ds(g * 16, 16)] = bvs[g]
        pidx[pl.ds(g * 16, 16)] = bis[g]
    pltpu.sync_copy(pval, shval.at[pl.ds(s * C, C)])
    pltpu.sync_copy(pidx, shidx.at[pl.ds(s * C, C)])
    plsc.subcore_barrier()

    @pl.when(q == 0)
    def _():
        pltpu.sync_copy(shval.at[pl.ds(s * C, WPB * C)], cmbv)
        pltpu.sync_copy(shidx.at[pl.ds(s * C, WPB * C)], cmbi)
        for g in range(G):
            bv = cmbv[pl.ds(g * 16, 16)]
            bi = cmbi[pl.ds(g * 16, 16)]
            for j in range(1, WPB):
                v = cmbv[pl.ds(j * C + g * 16, 16)]
                i = cmbi[pl.ds(j * C + g * 16, 16)]
                m = v > bv
                bv = jnp.where(m, v, bv)
                bi = jnp.where(m, i, bi)
            t = lax.shift_right_logical(bi, 7)
            y = lax.shift_right_logical(t * 21846, 16)
            x = bi - y * W
            obuf[pl.ds(g * 16, 16)] = y.astype(jnp.float32)
            obuf[pl.ds(C + g * 16, 16)] = x.astype(jnp.float32)
        pltpu.sync_copy(obuf, out_hbm.at[b])


@jax.jit
def kernel(inputs):
    flat = jnp.reshape(inputs, (B * HW * C,))
    run = pl.kernel(
        _argmax_body,
        out_type=jax.ShapeDtypeStruct((B, 2 * C), jnp.float32),
        mesh=plsc.VectorSubcoreMesh(core_axis_name="c", subcore_axis_name="s"),
        scratch_types=(
            [pltpu.VMEM((CH,), jnp.float32) for _ in range(NB)]
            + [pltpu.VMEM((C,), jnp.float32), pltpu.VMEM((C,), jnp.int32),
               pltpu.VMEM_SHARED((NS * C,), jnp.float32),
               pltpu.VMEM_SHARED((NS * C,), jnp.int32),
               pltpu.VMEM((WPB * C,), jnp.float32),
               pltpu.VMEM((WPB * C,), jnp.int32),
               pltpu.VMEM((2 * C,), jnp.float32)]
            + [pltpu.SemaphoreType.DMA for _ in range(NB)]
        ),
    )
    return jnp.reshape(run(flat), (B, 2, C))
